# baseline (speedup 1.0000x reference)
import math

import numpy as np
import jax
import jax.numpy as jnp
from jax.experimental import pallas as pl
from jax.experimental.pallas import tpu as pltpu

_MAX_LEN = 10000
_D = 512
_B = 16384
_BLK = 1024

_div_term = np.exp(np.arange(0, _D, 2, dtype=np.float32) * (-np.log(10000.0) / _D))
_DIV_FULL = np.repeat(_div_term, 2).reshape(1, _D)
_PHASE = np.tile(np.array([0.0, math.pi / 2], dtype=np.float32), _D // 2).reshape(1, _D)


def _tc_body(t_ref, div_ref, phase_ref, out_ref):
    ang = t_ref[:, :] * div_ref[:, :] + phase_ref[:, :]
    out_ref[:, :] = jnp.sin(ang)


@jax.jit
def kernel(timesteps, pos_encoding):
    del pos_encoding
    t = timesteps.astype(jnp.float32).reshape(_B, 1)
    return pl.pallas_call(
        _tc_body,
        grid=(_B // _BLK,),
        in_specs=[
            pl.BlockSpec((_BLK, 1), lambda i: (i, 0)),
            pl.BlockSpec((1, _D), lambda i: (0, 0)),
            pl.BlockSpec((1, _D), lambda i: (0, 0)),
        ],
        out_specs=pl.BlockSpec((_BLK, _D), lambda i: (i, 0)),
        out_shape=jax.ShapeDtypeStruct((_B, _D), jnp.float32),
    )(t, jnp.asarray(_DIV_FULL), jnp.asarray(_PHASE))


# speedup vs baseline: 2.7183x; 2.7183x over previous
import functools

import jax
import jax.numpy as jnp
from jax import lax
from jax.experimental import pallas as pl
from jax.experimental.pallas import tpu as pltpu
from jax.experimental.pallas import tpu_sc as plsc

_MAX_LEN = 10000
_D = 512
_B = 16384

_info = plsc.get_sparse_core_info()
_NC = _info.num_cores
_NS = _info.num_subcores
_NW = _NC * _NS
_B_PER_W = _B // _NW
_CHUNK = 32
_NCHUNK = _B_PER_W // _CHUNK
_NBUF = 6


def _gather_body(table_hbm, idx_hbm, out_hbm, idx_v, rows_v, gsem, ssem):
    wid = lax.axis_index("s") * _NC + lax.axis_index("c")
    base = wid * _B_PER_W
    pltpu.sync_copy(idx_hbm.at[pl.ds(base, _B_PER_W)], idx_v)

    def start_gather(c):
        return pltpu.async_copy(
            table_hbm.at[idx_v.at[pl.ds(c * _CHUNK, _CHUNK)]],
            rows_v.at[c % _NBUF],
            gsem.at[c % _NBUF],
        )

    def start_store(c):
        return pltpu.async_copy(
            rows_v.at[c % _NBUF],
            out_hbm.at[pl.ds(base + c * _CHUNK, _CHUNK)],
            ssem.at[c % _NBUF],
        )

    gathers = [None] * _NCHUNK
    stores = [None] * _NCHUNK
    for c in range(_NBUF):
        gathers[c] = start_gather(c)
    for c in range(_NCHUNK):
        gathers[c].wait()
        stores[c] = start_store(c)
        if c + _NBUF < _NCHUNK:
            stores[c].wait()
            gathers[c + _NBUF] = start_gather(c + _NBUF)
    for c in range(_NCHUNK - _NBUF, _NCHUNK):
        stores[c].wait()


@jax.jit
def kernel(timesteps, pos_encoding):
    mesh = plsc.VectorSubcoreMesh(core_axis_name="c", subcore_axis_name="s")
    run = functools.partial(
        pl.kernel,
        mesh=mesh,
        out_type=jax.ShapeDtypeStruct((_B, _D), jnp.float32),
        scratch_types=[
            pltpu.VMEM((_B_PER_W,), jnp.int32),
            pltpu.VMEM((_NBUF, _CHUNK, _D), jnp.float32),
            pltpu.SemaphoreType.DMA((_NBUF,)),
            pltpu.SemaphoreType.DMA((_NBUF,)),
        ],
    )(_gather_body)
    return run(pos_encoding, timesteps.astype(jnp.int32))
